# baseline (speedup 1.0000x reference)
import functools

import jax
import jax.numpy as jnp
from jax import lax
from jax.experimental import pallas as pl
from jax.experimental.pallas import tpu as pltpu
from jax.experimental.pallas import tpu_sc as plsc

_B = 2
_S = 2048
_N = _B * _S
_D = 1024
_F = 4096
_E = 8

_TB = 512
_NT = _N // _TB

_BLKR = 128
_L = 8192 + _E * _BLKR
_NR = _L // _BLKR

_NW = 32
_TPW = _N // _NW
_CH = 32
_CC = 16


def _router_body(x_ref, rw_ref, rb_ref,
                 i1_ref, i2_ref, r1_ref, r2_ref, w1_ref, w2_ref,
                 cnt_ref, aux_ref, carry_ref, gsum_ref):
    t = pl.program_id(0)

    xb = x_ref[...]
    logits = jnp.dot(xb, rw_ref[...], preferred_element_type=jnp.float32)
    logits = logits + rb_ref[...]
    logits = logits - jnp.max(logits, axis=1, keepdims=True)
    eg = jnp.exp(logits)
    gate = eg / jnp.sum(eg, axis=1, keepdims=True)

    lanes = lax.broadcasted_iota(jnp.int32, (_TB, _E), 1)
    m1 = jnp.max(gate, axis=1, keepdims=True)
    i1 = jnp.min(jnp.where(gate == m1, lanes, _E), axis=1, keepdims=True)
    oh1 = (lanes == i1).astype(jnp.float32)
    gate_wo = jnp.where(lanes == i1, -jnp.inf, gate)
    m2 = jnp.max(gate_wo, axis=1, keepdims=True)
    i2 = jnp.min(jnp.where(gate_wo == m2, lanes, _E), axis=1, keepdims=True)
    oh2 = (lanes == i2).astype(jnp.float32)
    m12 = oh1 + oh2

    @pl.when(t == 0)
    def _():
        carry_ref[...] = jnp.zeros((1, _E), jnp.float32)
        gsum_ref[...] = jnp.zeros((1, _E), jnp.float32)

    carry = carry_ref[...]

    rows = lax.broadcasted_iota(jnp.int32, (_TB, _TB), 0)
    cols = lax.broadcasted_iota(jnp.int32, (_TB, _TB), 1)
    ltri = (rows > cols).astype(jnp.float32)
    excl = jnp.dot(ltri, m12, preferred_element_type=jnp.float32)
    tot = excl + carry

    rank1 = jnp.sum(oh1 * tot, axis=1, keepdims=True)
    rank2 = jnp.sum(oh2 * (tot + oh1), axis=1, keepdims=True)

    i1_ref[0] = i1
    i2_ref[0] = i2
    r1_ref[0] = rank1.astype(jnp.int32)
    r2_ref[0] = rank2.astype(jnp.int32)
    w1_ref[0] = m1
    w2_ref[0] = m2

    carry_new = carry + jnp.sum(m12, axis=0, keepdims=True)
    carry_ref[...] = carry_new
    gsum_ref[...] += jnp.sum(gate, axis=0, keepdims=True)

    @pl.when(t == _NT - 1)
    def _():
        cnt_ref[...] = carry_new
        p = gsum_ref[...] / float(_N)
        aux_ref[...] = jnp.sum(p * jnp.log(p + 1e-9), keepdims=True
                               ).reshape(1, 1)


def _run_router(x2d, router_W, rb2d):
    col = lambda dt: jax.ShapeDtypeStruct((_NT, _TB, 1), dt)
    col_spec = pl.BlockSpec((1, _TB, 1), lambda t: (t, 0, 0))
    return pl.pallas_call(
        _router_body,
        grid=(_NT,),
        in_specs=[
            pl.BlockSpec((_TB, _D), lambda t: (t, 0)),
            pl.BlockSpec((_D, _E), lambda t: (0, 0)),
            pl.BlockSpec((1, _E), lambda t: (0, 0)),
        ],
        out_specs=[col_spec, col_spec, col_spec, col_spec, col_spec,
                   col_spec,
                   pl.BlockSpec((1, _E), lambda t: (0, 0)),
                   pl.BlockSpec((1, 1), lambda t: (0, 0))],
        out_shape=[col(jnp.int32), col(jnp.int32), col(jnp.int32),
                   col(jnp.int32), col(jnp.float32), col(jnp.float32),
                   jax.ShapeDtypeStruct((1, _E), jnp.float32),
                   jax.ShapeDtypeStruct((1, 1), jnp.float32)],
        scratch_shapes=[pltpu.VMEM((1, _E), jnp.float32),
                        pltpu.VMEM((1, _E), jnp.float32)],
        compiler_params=pltpu.CompilerParams(
            dimension_semantics=("arbitrary",)),
    )(x2d, router_W, rb2d)


def _slots_body(i1_ref, r1_ref, i2_ref, r2_ref, offs_ref, s1_ref, s2_ref):
    offs = offs_ref[...].astype(jnp.float32)
    lanes = lax.broadcasted_iota(jnp.int32, (_TB, _E), 1)

    def slot(i_ref, r_ref):
        oh = (lanes == i_ref[0]).astype(jnp.float32)
        off = jnp.sum(oh * offs, axis=1, keepdims=True)
        return off.astype(jnp.int32) + r_ref[0]

    s1_ref[0] = slot(i1_ref, r1_ref)
    s2_ref[0] = slot(i2_ref, r2_ref)


def _run_slots(i1o, r1o, i2o, r2o, offs2d):
    col_spec = pl.BlockSpec((1, _TB, 1), lambda t: (t, 0, 0))
    col = jax.ShapeDtypeStruct((_NT, _TB, 1), jnp.int32)
    return pl.pallas_call(
        _slots_body,
        grid=(_NT,),
        in_specs=[col_spec, col_spec, col_spec, col_spec,
                  pl.BlockSpec((1, _E), lambda t: (0, 0))],
        out_specs=[col_spec, col_spec],
        out_shape=[col, col],
    )(i1o, r1o, i2o, r2o, offs2d)


@functools.lru_cache(maxsize=None)
def _make_dispatch():
    mesh = plsc.VectorSubcoreMesh(core_axis_name="c", subcore_axis_name="s")

    @functools.partial(
        pl.kernel, mesh=mesh,
        out_type=jax.ShapeDtypeStruct((_L, _D), jnp.float32),
        scratch_types=[
            pltpu.VMEM((2 * (_TPW // _CH), _CH), jnp.int32),
            pltpu.VMEM((_CH, _D), jnp.float32),
            pltpu.VMEM((_CH, _D), jnp.float32),
            pltpu.SemaphoreType.DMA,
            pltpu.SemaphoreType.DMA,
            pltpu.SemaphoreType.DMA,
        ],
    )
    def dispatch(x_hbm, sall_hbm, xs_hbm,
                 slot_v, xb0, xb1, sem0, sem1, semi):
        wid = lax.axis_index("s") * 2 + lax.axis_index("c")
        base = wid * _TPW
        nch = _TPW // _CH
        pltpu.async_copy(sall_hbm.at[wid], slot_v, semi).wait()
        xb = (xb0, xb1)
        sems = (sem0, sem1)
        pend = [[], []]
        for c in range(nch):
            b = c & 1
            for h in pend[b]:
                h.wait()
            pltpu.sync_copy(x_hbm.at[pl.ds(base + c * _CH, _CH)], xb[b])
            h1 = pltpu.async_copy(xb[b], xs_hbm.at[slot_v.at[c]], sems[b])
            h2 = pltpu.async_copy(xb[b], xs_hbm.at[slot_v.at[nch + c]],
                                  sems[b])
            pend[b] = [h1, h2]
        for b in (0, 1):
            for h in pend[b]:
                h.wait()

    return dispatch


def _gmm_body(eid_ref, xs_ref, w1_hbm, b1_ref, w2_hbm, b2_ref, ys_ref,
              w1f, w2f, w1b, w2b, sem1, sem2):
    r = pl.program_id(0)
    e = eid_ref[r]
    prev = eid_ref[jnp.maximum(r - 1, 0)]
    nxt = eid_ref[jnp.minimum(r + 1, _NR - 1)]

    @pl.when(r == 0)
    def _start_first():
        pltpu.make_async_copy(w1_hbm.at[e], w1f, sem1).start()
        pltpu.make_async_copy(w2_hbm.at[e], w2f, sem2).start()

    @pl.when((r == 0) | (e != prev))
    def _convert_weights():
        pltpu.make_async_copy(w1_hbm.at[e], w1f, sem1).wait()
        w1b[...] = w1f[...].astype(jnp.bfloat16)
        pltpu.make_async_copy(w2_hbm.at[e], w2f, sem2).wait()
        w2b[...] = w2f[...].astype(jnp.bfloat16)

    @pl.when((r < _NR - 1) & (nxt != e))
    def _prefetch_next():
        pltpu.make_async_copy(w1_hbm.at[nxt], w1f, sem1).start()
        pltpu.make_async_copy(w2_hbm.at[nxt], w2f, sem2).start()

    h = jnp.dot(xs_ref[...].astype(jnp.bfloat16), w1b[...],
                preferred_element_type=jnp.float32)
    h = jnp.maximum(h + b1_ref[0], 0.0)
    y = jnp.dot(h.astype(jnp.bfloat16), w2b[...],
                preferred_element_type=jnp.float32)
    ys_ref[...] = y + b2_ref[0]


def _run_gmm(eid, xs, w1, b1r, w2, b2r):
    return pl.pallas_call(
        _gmm_body,
        grid_spec=pltpu.PrefetchScalarGridSpec(
            num_scalar_prefetch=1,
            grid=(_NR,),
            in_specs=[
                pl.BlockSpec((_BLKR, _D), lambda r, eid: (r, 0)),
                pl.BlockSpec(memory_space=pl.ANY),
                pl.BlockSpec((1, 1, _F), lambda r, eid: (eid[r], 0, 0)),
                pl.BlockSpec(memory_space=pl.ANY),
                pl.BlockSpec((1, 1, _D), lambda r, eid: (eid[r], 0, 0)),
            ],
            out_specs=pl.BlockSpec((_BLKR, _D), lambda r, eid: (r, 0)),
            scratch_shapes=[
                pltpu.VMEM((_D, _F), jnp.float32),
                pltpu.VMEM((_F, _D), jnp.float32),
                pltpu.VMEM((_D, _F), jnp.bfloat16),
                pltpu.VMEM((_F, _D), jnp.bfloat16),
                pltpu.SemaphoreType.DMA,
                pltpu.SemaphoreType.DMA,
            ],
        ),
        out_shape=jax.ShapeDtypeStruct((_L, _D), jnp.float32),
        compiler_params=pltpu.CompilerParams(
            dimension_semantics=("arbitrary",)),
    )(eid, xs, w1, b1r, w2, b2r)


@functools.lru_cache(maxsize=None)
def _make_combine():
    mesh = plsc.VectorSubcoreMesh(core_axis_name="c", subcore_axis_name="s")

    cc = _CC
    nch = _TPW // cc

    @functools.partial(
        pl.kernel, mesh=mesh,
        out_type=[jax.ShapeDtypeStruct((_N, _D), jnp.float32),
                  jax.ShapeDtypeStruct((_N, _D), jnp.float32)],
        scratch_types=[
            pltpu.VMEM((2 * nch, cc), jnp.int32),
            pltpu.VMEM((cc, _D), jnp.float32),
            pltpu.VMEM((cc, _D), jnp.float32),
            pltpu.VMEM((cc, _D), jnp.float32),
            pltpu.VMEM((cc, _D), jnp.float32),
            pltpu.SemaphoreType.DMA,
            pltpu.SemaphoreType.DMA,
            pltpu.SemaphoreType.DMA,
            pltpu.SemaphoreType.DMA,
            pltpu.SemaphoreType.DMA,
        ],
    )
    def combine(ys_hbm, sall_hbm, y1_hbm, y2_hbm,
                slot_v, a0, a1, b0, b1, semg0, semg1, sems0, sems1, semi):
        wid = lax.axis_index("s") * 2 + lax.axis_index("c")
        base = wid * _TPW
        pltpu.async_copy(sall_hbm.at[wid], slot_v, semi).wait()
        av = (a0, a1)
        bv = (b0, b1)
        semg = (semg0, semg1)
        sems = (sems0, sems1)
        pend = [[], []]
        for c in range(nch):
            b = c & 1
            for h in pend[b]:
                h.wait()
            g1 = pltpu.async_copy(ys_hbm.at[slot_v.at[c]], av[b], semg[b])
            g2 = pltpu.async_copy(ys_hbm.at[slot_v.at[nch + c]], bv[b],
                                  semg[b])
            g1.wait()
            g2.wait()
            tb = base + c * cc
            h1 = pltpu.async_copy(av[b], y1_hbm.at[pl.ds(tb, cc)], sems[b])
            h2 = pltpu.async_copy(bv[b], y2_hbm.at[pl.ds(tb, cc)], sems[b])
            pend[b] = [h1, h2]
        for b in (0, 1):
            for h in pend[b]:
                h.wait()

    return combine


def _wadd_body(y1_ref, y2_ref, w1_ref, w2_ref, out_ref):
    out_ref[...] = y1_ref[...] * w1_ref[0] + y2_ref[...] * w2_ref[0]


def _run_wadd(y1g, y2g, w1o, w2o):
    col_spec = pl.BlockSpec((1, _TB, 1), lambda t: (t, 0, 0))
    return pl.pallas_call(
        _wadd_body,
        grid=(_NT,),
        in_specs=[pl.BlockSpec((_TB, _D), lambda t: (t, 0)),
                  pl.BlockSpec((_TB, _D), lambda t: (t, 0)),
                  col_spec, col_spec],
        out_specs=pl.BlockSpec((_TB, _D), lambda t: (t, 0)),
        out_shape=jax.ShapeDtypeStruct((_N, _D), jnp.float32),
    )(y1g, y2g, w1o, w2o)


@jax.jit
def kernel(x, router_W, router_b, W1, b1, W2, b2):
    x2d = x.reshape(_N, _D)
    rb2d = router_b.reshape(1, _E)

    (i1o, i2o, r1o, r2o, w1o, w2o, cnt, aux) = _run_router(
        x2d, router_W, rb2d)
    counts = cnt.reshape(_E).astype(jnp.int32)
    padded = ((counts + _BLKR - 1) // _BLKR) * _BLKR
    ends = jnp.cumsum(padded)
    offs = (ends - padded).astype(jnp.int32)
    rstart = jnp.arange(_NR, dtype=jnp.int32) * _BLKR
    eid = jnp.minimum(jnp.sum(rstart[:, None] >= ends[None, :], axis=1),
                      _E - 1).astype(jnp.int32)

    s1o, s2o = _run_slots(i1o, r1o, i2o, r2o, offs.reshape(1, _E))
    s1 = s1o.reshape(_N)
    s2 = s2o.reshape(_N)

    nd = _TPW // _CH
    sall_d = jnp.concatenate([s1.reshape(_NW, nd, _CH),
                              s2.reshape(_NW, nd, _CH)], axis=1)
    xs = _make_dispatch()(x2d, sall_d)

    ys = _run_gmm(eid, xs,
                  W1, b1.reshape(_E, 1, _F),
                  W2, b2.reshape(_E, 1, _D))

    nc = _TPW // _CC
    sall_c = jnp.concatenate([s1.reshape(_NW, nc, _CC),
                              s2.reshape(_NW, nc, _CC)], axis=1)
    y1g, y2g = _make_combine()(ys, sall_c)
    out2d = _run_wadd(y1g, y2g, w1o, w2o)
    return out2d.reshape(_B, _S, _D), aux.reshape(())


# speedup vs baseline: 1.0263x; 1.0263x over previous
import functools

import jax
import jax.numpy as jnp
from jax import lax
from jax.experimental import pallas as pl
from jax.experimental.pallas import tpu as pltpu
from jax.experimental.pallas import tpu_sc as plsc

_B = 2
_S = 2048
_N = _B * _S
_D = 1024
_F = 4096
_E = 8

_TB = 512
_NT = _N // _TB

_BLKR = 256
_L = 8192 + _E * _BLKR
_NR = _L // _BLKR

_NW = 32
_TPW = _N // _NW
_CH = 32
_CC = 16


def _router_body(x_ref, rw_ref, rb_ref,
                 i1_ref, i2_ref, r1_ref, r2_ref, w1_ref, w2_ref,
                 cnt_ref, aux_ref, carry_ref, gsum_ref):
    t = pl.program_id(0)

    xb = x_ref[...]
    logits = jnp.dot(xb, rw_ref[...], preferred_element_type=jnp.float32)
    logits = logits + rb_ref[...]
    logits = logits - jnp.max(logits, axis=1, keepdims=True)
    eg = jnp.exp(logits)
    gate = eg / jnp.sum(eg, axis=1, keepdims=True)

    lanes = lax.broadcasted_iota(jnp.int32, (_TB, _E), 1)
    m1 = jnp.max(gate, axis=1, keepdims=True)
    i1 = jnp.min(jnp.where(gate == m1, lanes, _E), axis=1, keepdims=True)
    oh1 = (lanes == i1).astype(jnp.float32)
    gate_wo = jnp.where(lanes == i1, -jnp.inf, gate)
    m2 = jnp.max(gate_wo, axis=1, keepdims=True)
    i2 = jnp.min(jnp.where(gate_wo == m2, lanes, _E), axis=1, keepdims=True)
    oh2 = (lanes == i2).astype(jnp.float32)
    m12 = oh1 + oh2

    @pl.when(t == 0)
    def _():
        carry_ref[...] = jnp.zeros((1, _E), jnp.float32)
        gsum_ref[...] = jnp.zeros((1, _E), jnp.float32)

    carry = carry_ref[...]

    rows = lax.broadcasted_iota(jnp.int32, (_TB, _TB), 0)
    cols = lax.broadcasted_iota(jnp.int32, (_TB, _TB), 1)
    ltri = (rows > cols).astype(jnp.float32)
    excl = jnp.dot(ltri, m12, preferred_element_type=jnp.float32)
    tot = excl + carry

    rank1 = jnp.sum(oh1 * tot, axis=1, keepdims=True)
    rank2 = jnp.sum(oh2 * (tot + oh1), axis=1, keepdims=True)

    i1_ref[0] = i1
    i2_ref[0] = i2
    r1_ref[0] = rank1.astype(jnp.int32)
    r2_ref[0] = rank2.astype(jnp.int32)
    w1_ref[0] = m1
    w2_ref[0] = m2

    carry_new = carry + jnp.sum(m12, axis=0, keepdims=True)
    carry_ref[...] = carry_new
    gsum_ref[...] += jnp.sum(gate, axis=0, keepdims=True)

    @pl.when(t == _NT - 1)
    def _():
        cnt_ref[...] = carry_new
        p = gsum_ref[...] / float(_N)
        aux_ref[...] = jnp.sum(p * jnp.log(p + 1e-9), keepdims=True
                               ).reshape(1, 1)


def _run_router(x2d, router_W, rb2d):
    col = lambda dt: jax.ShapeDtypeStruct((_NT, _TB, 1), dt)
    col_spec = pl.BlockSpec((1, _TB, 1), lambda t: (t, 0, 0))
    return pl.pallas_call(
        _router_body,
        grid=(_NT,),
        in_specs=[
            pl.BlockSpec((_TB, _D), lambda t: (t, 0)),
            pl.BlockSpec((_D, _E), lambda t: (0, 0)),
            pl.BlockSpec((1, _E), lambda t: (0, 0)),
        ],
        out_specs=[col_spec, col_spec, col_spec, col_spec, col_spec,
                   col_spec,
                   pl.BlockSpec((1, _E), lambda t: (0, 0)),
                   pl.BlockSpec((1, 1), lambda t: (0, 0))],
        out_shape=[col(jnp.int32), col(jnp.int32), col(jnp.int32),
                   col(jnp.int32), col(jnp.float32), col(jnp.float32),
                   jax.ShapeDtypeStruct((1, _E), jnp.float32),
                   jax.ShapeDtypeStruct((1, 1), jnp.float32)],
        scratch_shapes=[pltpu.VMEM((1, _E), jnp.float32),
                        pltpu.VMEM((1, _E), jnp.float32)],
        compiler_params=pltpu.CompilerParams(
            dimension_semantics=("arbitrary",)),
    )(x2d, router_W, rb2d)


def _slots_body(i1_ref, r1_ref, i2_ref, r2_ref, offs_ref, s1_ref, s2_ref):
    offs = offs_ref[...].astype(jnp.float32)
    lanes = lax.broadcasted_iota(jnp.int32, (_TB, _E), 1)

    def slot(i_ref, r_ref):
        oh = (lanes == i_ref[0]).astype(jnp.float32)
        off = jnp.sum(oh * offs, axis=1, keepdims=True)
        return off.astype(jnp.int32) + r_ref[0]

    s1_ref[0] = slot(i1_ref, r1_ref)
    s2_ref[0] = slot(i2_ref, r2_ref)


def _run_slots(i1o, r1o, i2o, r2o, offs2d):
    col_spec = pl.BlockSpec((1, _TB, 1), lambda t: (t, 0, 0))
    col = jax.ShapeDtypeStruct((_NT, _TB, 1), jnp.int32)
    return pl.pallas_call(
        _slots_body,
        grid=(_NT,),
        in_specs=[col_spec, col_spec, col_spec, col_spec,
                  pl.BlockSpec((1, _E), lambda t: (0, 0))],
        out_specs=[col_spec, col_spec],
        out_shape=[col, col],
    )(i1o, r1o, i2o, r2o, offs2d)


@functools.lru_cache(maxsize=None)
def _make_dispatch():
    mesh = plsc.VectorSubcoreMesh(core_axis_name="c", subcore_axis_name="s")

    @functools.partial(
        pl.kernel, mesh=mesh,
        out_type=jax.ShapeDtypeStruct((_L, _D), jnp.float32),
        scratch_types=[
            pltpu.VMEM((2 * (_TPW // _CH), _CH), jnp.int32),
            pltpu.VMEM((_CH, _D), jnp.float32),
            pltpu.VMEM((_CH, _D), jnp.float32),
            pltpu.SemaphoreType.DMA,
            pltpu.SemaphoreType.DMA,
            pltpu.SemaphoreType.DMA,
        ],
    )
    def dispatch(x_hbm, sall_hbm, xs_hbm,
                 slot_v, xb0, xb1, sem0, sem1, semi):
        wid = lax.axis_index("s") * 2 + lax.axis_index("c")
        base = wid * _TPW
        nch = _TPW // _CH
        pltpu.async_copy(sall_hbm.at[wid], slot_v, semi).wait()
        xb = (xb0, xb1)
        sems = (sem0, sem1)
        pend = [[], []]
        for c in range(nch):
            b = c & 1
            for h in pend[b]:
                h.wait()
            pltpu.sync_copy(x_hbm.at[pl.ds(base + c * _CH, _CH)], xb[b])
            h1 = pltpu.async_copy(xb[b], xs_hbm.at[slot_v.at[c]], sems[b])
            h2 = pltpu.async_copy(xb[b], xs_hbm.at[slot_v.at[nch + c]],
                                  sems[b])
            pend[b] = [h1, h2]
        for b in (0, 1):
            for h in pend[b]:
                h.wait()

    return dispatch


def _gmm_body(eid_ref, xs_ref, w1_hbm, b1_ref, w2_hbm, b2_ref, ys_ref,
              w1f, w2f, w1b, w2b, sem1, sem2):
    r = pl.program_id(0)
    e = eid_ref[r]
    prev = eid_ref[jnp.maximum(r - 1, 0)]
    nxt = eid_ref[jnp.minimum(r + 1, _NR - 1)]

    @pl.when(r == 0)
    def _start_first():
        pltpu.make_async_copy(w1_hbm.at[e], w1f, sem1).start()
        pltpu.make_async_copy(w2_hbm.at[e], w2f, sem2).start()

    @pl.when((r == 0) | (e != prev))
    def _convert_weights():
        pltpu.make_async_copy(w1_hbm.at[e], w1f, sem1).wait()
        w1b[...] = w1f[...].astype(jnp.bfloat16)
        pltpu.make_async_copy(w2_hbm.at[e], w2f, sem2).wait()
        w2b[...] = w2f[...].astype(jnp.bfloat16)

    @pl.when((r < _NR - 1) & (nxt != e))
    def _prefetch_next():
        pltpu.make_async_copy(w1_hbm.at[nxt], w1f, sem1).start()
        pltpu.make_async_copy(w2_hbm.at[nxt], w2f, sem2).start()

    h = jnp.dot(xs_ref[...].astype(jnp.bfloat16), w1b[...],
                preferred_element_type=jnp.float32)
    h = jnp.maximum(h + b1_ref[0], 0.0)
    y = jnp.dot(h.astype(jnp.bfloat16), w2b[...],
                preferred_element_type=jnp.float32)
    ys_ref[...] = y + b2_ref[0]


def _run_gmm(eid, xs, w1, b1r, w2, b2r):
    return pl.pallas_call(
        _gmm_body,
        grid_spec=pltpu.PrefetchScalarGridSpec(
            num_scalar_prefetch=1,
            grid=(_NR,),
            in_specs=[
                pl.BlockSpec((_BLKR, _D), lambda r, eid: (r, 0)),
                pl.BlockSpec(memory_space=pl.ANY),
                pl.BlockSpec((1, 1, _F), lambda r, eid: (eid[r], 0, 0)),
                pl.BlockSpec(memory_space=pl.ANY),
                pl.BlockSpec((1, 1, _D), lambda r, eid: (eid[r], 0, 0)),
            ],
            out_specs=pl.BlockSpec((_BLKR, _D), lambda r, eid: (r, 0)),
            scratch_shapes=[
                pltpu.VMEM((_D, _F), jnp.float32),
                pltpu.VMEM((_F, _D), jnp.float32),
                pltpu.VMEM((_D, _F), jnp.bfloat16),
                pltpu.VMEM((_F, _D), jnp.bfloat16),
                pltpu.SemaphoreType.DMA,
                pltpu.SemaphoreType.DMA,
            ],
        ),
        out_shape=jax.ShapeDtypeStruct((_L, _D), jnp.float32),
        compiler_params=pltpu.CompilerParams(
            dimension_semantics=("arbitrary",)),
    )(eid, xs, w1, b1r, w2, b2r)


@functools.lru_cache(maxsize=None)
def _make_combine():
    mesh = plsc.VectorSubcoreMesh(core_axis_name="c", subcore_axis_name="s")

    cc = _CC
    nch = _TPW // cc

    @functools.partial(
        pl.kernel, mesh=mesh,
        out_type=[jax.ShapeDtypeStruct((_N, _D), jnp.float32),
                  jax.ShapeDtypeStruct((_N, _D), jnp.float32)],
        scratch_types=[
            pltpu.VMEM((2 * nch, cc), jnp.int32),
            pltpu.VMEM((cc, _D), jnp.float32),
            pltpu.VMEM((cc, _D), jnp.float32),
            pltpu.VMEM((cc, _D), jnp.float32),
            pltpu.VMEM((cc, _D), jnp.float32),
            pltpu.SemaphoreType.DMA,
            pltpu.SemaphoreType.DMA,
            pltpu.SemaphoreType.DMA,
            pltpu.SemaphoreType.DMA,
            pltpu.SemaphoreType.DMA,
        ],
    )
    def combine(ys_hbm, sall_hbm, y1_hbm, y2_hbm,
                slot_v, a0, a1, b0, b1, semg0, semg1, sems0, sems1, semi):
        wid = lax.axis_index("s") * 2 + lax.axis_index("c")
        base = wid * _TPW
        pltpu.async_copy(sall_hbm.at[wid], slot_v, semi).wait()
        av = (a0, a1)
        bv = (b0, b1)
        semg = (semg0, semg1)
        sems = (sems0, sems1)
        pend = [[], []]
        for c in range(nch):
            b = c & 1
            for h in pend[b]:
                h.wait()
            g1 = pltpu.async_copy(ys_hbm.at[slot_v.at[c]], av[b], semg[b])
            g2 = pltpu.async_copy(ys_hbm.at[slot_v.at[nch + c]], bv[b],
                                  semg[b])
            g1.wait()
            g2.wait()
            tb = base + c * cc
            h1 = pltpu.async_copy(av[b], y1_hbm.at[pl.ds(tb, cc)], sems[b])
            h2 = pltpu.async_copy(bv[b], y2_hbm.at[pl.ds(tb, cc)], sems[b])
            pend[b] = [h1, h2]
        for b in (0, 1):
            for h in pend[b]:
                h.wait()

    return combine


def _wadd_body(y1_ref, y2_ref, w1_ref, w2_ref, out_ref):
    out_ref[...] = y1_ref[...] * w1_ref[0] + y2_ref[...] * w2_ref[0]


def _run_wadd(y1g, y2g, w1o, w2o):
    col_spec = pl.BlockSpec((1, _TB, 1), lambda t: (t, 0, 0))
    return pl.pallas_call(
        _wadd_body,
        grid=(_NT,),
        in_specs=[pl.BlockSpec((_TB, _D), lambda t: (t, 0)),
                  pl.BlockSpec((_TB, _D), lambda t: (t, 0)),
                  col_spec, col_spec],
        out_specs=pl.BlockSpec((_TB, _D), lambda t: (t, 0)),
        out_shape=jax.ShapeDtypeStruct((_N, _D), jnp.float32),
    )(y1g, y2g, w1o, w2o)


@jax.jit
def kernel(x, router_W, router_b, W1, b1, W2, b2):
    x2d = x.reshape(_N, _D)
    rb2d = router_b.reshape(1, _E)

    (i1o, i2o, r1o, r2o, w1o, w2o, cnt, aux) = _run_router(
        x2d, router_W, rb2d)
    counts = cnt.reshape(_E).astype(jnp.int32)
    padded = ((counts + _BLKR - 1) // _BLKR) * _BLKR
    ends = jnp.cumsum(padded)
    offs = (ends - padded).astype(jnp.int32)
    rstart = jnp.arange(_NR, dtype=jnp.int32) * _BLKR
    eid = jnp.minimum(jnp.sum(rstart[:, None] >= ends[None, :], axis=1),
                      _E - 1).astype(jnp.int32)

    s1o, s2o = _run_slots(i1o, r1o, i2o, r2o, offs.reshape(1, _E))
    s1 = s1o.reshape(_N)
    s2 = s2o.reshape(_N)

    nd = _TPW // _CH
    sall_d = jnp.concatenate([s1.reshape(_NW, nd, _CH),
                              s2.reshape(_NW, nd, _CH)], axis=1)
    xs = _make_dispatch()(x2d, sall_d)

    ys = _run_gmm(eid, xs,
                  W1, b1.reshape(_E, 1, _F),
                  W2, b2.reshape(_E, 1, _D))

    nc = _TPW // _CC
    sall_c = jnp.concatenate([s1.reshape(_NW, nc, _CC),
                              s2.reshape(_NW, nc, _CC)], axis=1)
    y1g, y2g = _make_combine()(ys, sall_c)
    out2d = _run_wadd(y1g, y2g, w1o, w2o)
    return out2d.reshape(_B, _S, _D), aux.reshape(())
